# Initial kernel scaffold; baseline (speedup 1.0000x reference)
import jax
import jax.numpy as jnp
from jax.experimental import pallas as pl


def kernel(input_atom, input_bond, atom_graph, bond_graph, num_nbs, W_atom, W_U2, b_U2, W_U1, b_U1):
    raise NotImplementedError("write your pallas kernel here")


# speedup vs baseline: 1.0105x; 1.0105x over previous
import functools

import jax
import jax.numpy as jnp
from jax import lax
from jax.experimental import pallas as pl
from jax.experimental.pallas import tpu as pltpu
from jax.experimental.pallas import tpu_sc as plsc

B, N, E, MAX_NB, H, F_ATOM, F_BOND, DEPTH = 4, 12500, 12500, 10, 128, 89, 5, 3
BN = B * N
BE = B * E
NW = 32
CROWS = 8
NP = 50688
RPW = NP // NW
CG = CROWS * MAX_NB
NPE = 50176
SENT = BE
NEG = -1e30
TM = 512


def _mm_init_body(x_ref, wat_ref, w2a_ref, b2_ref, af_ref, a_ref):
    af = jnp.dot(x_ref[...], wat_ref[...], preferred_element_type=jnp.float32)
    af_ref[...] = af
    a_ref[...] = jnp.dot(af, w2a_ref[...], preferred_element_type=jnp.float32) + b2_ref[...]


def _mm_step_body(af_ref, nei_ref, w1a_ref, w1b_ref, b1_ref, w2a_ref, b2_ref,
                  af_out, a_out):
    x = jnp.dot(af_ref[...], w1a_ref[...], preferred_element_type=jnp.float32)
    x = x + jnp.dot(nei_ref[...], w1b_ref[...], preferred_element_type=jnp.float32)
    x = jnp.maximum(x + b1_ref[...], 0.0)
    af_out[...] = x
    a_out[...] = jnp.dot(x, w2a_ref[...], preferred_element_type=jnp.float32) + b2_ref[...]


def _mm_final_body(af_ref, nei_ref, w1a_ref, w1b_ref, b1_ref, af_out):
    x = jnp.dot(af_ref[...], w1a_ref[...], preferred_element_type=jnp.float32)
    x = x + jnp.dot(nei_ref[...], w1b_ref[...], preferred_element_type=jnp.float32)
    af_out[...] = jnp.maximum(x + b1_ref[...], 0.0)


def _mm_bond_body(bond_ref, w2b_ref, out_ref):
    v = jnp.dot(bond_ref[...], w2b_ref[...], preferred_element_type=jnp.float32)
    row = pl.program_id(0) * TM + lax.broadcasted_iota(jnp.int32, (TM, H), 0)
    out_ref[...] = jnp.where(row < BE, v, NEG)


def _full(shape):
    return pl.BlockSpec(shape, lambda i: (0, 0))


def _rows(w):
    return pl.BlockSpec((TM, w), lambda i: (i, 0))


def _tc_init(xp, wat, w2a, b2):
    return pl.pallas_call(
        _mm_init_body,
        grid=(NP // TM,),
        in_specs=[_rows(128), _full((128, H)), _full((H, H)), _full((1, H))],
        out_specs=[_rows(H), _rows(H)],
        out_shape=[jax.ShapeDtypeStruct((NP, H), jnp.float32),
                   jax.ShapeDtypeStruct((NP, H), jnp.float32)],
    )(xp, wat, w2a, b2)


def _tc_step(af, nei, w1a, w1b, b1, w2a, b2):
    return pl.pallas_call(
        _mm_step_body,
        grid=(NP // TM,),
        in_specs=[_rows(H), _rows(H), _full((H, H)), _full((H, H)), _full((1, H)),
                  _full((H, H)), _full((1, H))],
        out_specs=[_rows(H), _rows(H)],
        out_shape=[jax.ShapeDtypeStruct((NP, H), jnp.float32),
                   jax.ShapeDtypeStruct((NP, H), jnp.float32)],
    )(af, nei, w1a, w1b, b1, w2a, b2)


def _tc_final(af, nei, w1a, w1b, b1):
    return pl.pallas_call(
        _mm_final_body,
        grid=(NP // TM,),
        in_specs=[_rows(H), _rows(H), _full((H, H)), _full((H, H)), _full((1, H))],
        out_specs=_rows(H),
        out_shape=jax.ShapeDtypeStruct((NP, H), jnp.float32),
    )(af, nei, w1a, w1b, b1)


def _tc_bond(bondp, w2b):
    return pl.pallas_call(
        _mm_bond_body,
        grid=(NPE // TM,),
        in_specs=[pl.BlockSpec((TM, 8), lambda i: (i, 0)), _full((8, H))],
        out_specs=_rows(H),
        out_shape=jax.ShapeDtypeStruct((NPE, H), jnp.float32),
    )(bondp, w2b)


def _sc_nei(A, Bv, a_idx, e_idx):
    info = plsc.get_sparse_core_info()
    nc = info.num_cores
    mesh = plsc.VectorSubcoreMesh(core_axis_name="c", subcore_axis_name="s")

    @functools.partial(
        pl.kernel,
        mesh=mesh,
        out_type=jax.ShapeDtypeStruct((NP, H), jnp.float32),
        scratch_types=[
            pltpu.VMEM((CG,), jnp.int32),
            pltpu.VMEM((CG,), jnp.int32),
            pltpu.VMEM((CG, H), jnp.float32),
            pltpu.VMEM((CG, H), jnp.float32),
            pltpu.VMEM((CROWS, H), jnp.float32),
            pltpu.SemaphoreType.DMA,
            pltpu.SemaphoreType.DMA,
        ],
    )
    def k(a_hbm, bv_hbm, ai_hbm, ei_hbm, out_hbm, ai_v, ei_v, ga, gb, oc, s1, s2):
        wid = lax.axis_index("s") * nc + lax.axis_index("c")
        row_base = wid * RPW

        def chunk(ci, carry):
            r0 = row_base + ci * CROWS
            g0 = r0 * MAX_NB
            pltpu.sync_copy(ai_hbm.at[pl.ds(g0, CG)], ai_v)
            pltpu.sync_copy(ei_hbm.at[pl.ds(g0, CG)], ei_v)
            cp1 = pltpu.async_copy(a_hbm.at[ai_v], ga, s1)
            cp2 = pltpu.async_copy(bv_hbm.at[ei_v], gb, s2)
            cp1.wait()
            cp2.wait()

            def row(r, c2):
                rk = r * MAX_NB
                for h in range(H // 16):
                    sl = pl.ds(h * 16, 16)
                    acc = jnp.zeros((16,), jnp.float32)
                    for kk in range(MAX_NB):
                        v = ga[rk + kk, sl] + gb[rk + kk, sl]
                        acc = acc + jnp.maximum(v, 0.0)
                    oc[r, sl] = acc
                return c2

            lax.fori_loop(0, CROWS, row, 0)
            pltpu.sync_copy(oc, out_hbm.at[pl.ds(r0, CROWS)])
            return carry

        lax.fori_loop(0, RPW // CROWS, chunk, 0)

    return k(A, Bv, a_idx, e_idx)


def kernel(input_atom, input_bond, atom_graph, bond_graph, num_nbs,
           W_atom, W_U2, b_U2, W_U1, b_U1):
    f32 = jnp.float32
    x = input_atom.reshape(BN, F_ATOM)
    xp = jnp.zeros((NP, 128), f32).at[:BN, :F_ATOM].set(x)
    wat = jnp.zeros((128, H), f32).at[:F_ATOM].set(W_atom)
    bond = input_bond.reshape(BE, F_BOND)
    bondp = jnp.zeros((NPE, 8), f32).at[:BE, :F_BOND].set(bond)
    w2a = W_U2[:H]
    w2b = jnp.zeros((8, H), f32).at[:F_BOND].set(W_U2[H:])
    w1a = W_U1[:H]
    w1b = W_U1[H:]
    b1 = b_U1.reshape(1, H)
    b2 = b_U2.reshape(1, H)

    a_flat = (atom_graph[..., 0] * N + atom_graph[..., 1]).reshape(-1)
    valid = jnp.arange(MAX_NB, dtype=jnp.int32)[None, None, :] < num_nbs[:, :, None]
    e_flat = jnp.where(valid, bond_graph[..., 0] * E + bond_graph[..., 1],
                       SENT).reshape(-1)
    pad_g = NP * MAX_NB - BN * MAX_NB
    a_flat = jnp.concatenate([a_flat, jnp.zeros((pad_g,), jnp.int32)])
    e_flat = jnp.concatenate([e_flat, jnp.full((pad_g,), SENT, jnp.int32)])

    bv = _tc_bond(bondp, w2b)
    af, a_tab = _tc_init(xp, wat, w2a, b2)
    for d in range(DEPTH):
        nei = _sc_nei(a_tab, bv, a_flat, e_flat)
        if d < DEPTH - 1:
            af, a_tab = _tc_step(af, nei, w1a, w1b, b1, w2a, b2)
        else:
            af = _tc_final(af, nei, w1a, w1b, b1)
    return af[:BN].reshape(B, N, H)


# speedup vs baseline: 1.0128x; 1.0023x over previous
import functools

import jax
import jax.numpy as jnp
from jax import lax
from jax.experimental import pallas as pl
from jax.experimental.pallas import tpu as pltpu
from jax.experimental.pallas import tpu_sc as plsc

B, N, E, MAX_NB, H, F_ATOM, F_BOND, DEPTH = 4, 12500, 12500, 10, 128, 89, 5, 3
BN = B * N
BE = B * E
NW = 32
CROWS = 16
SUB = 80
NP = 50688
RPW = NP // NW
CG = CROWS * MAX_NB
NPE = 50176
SENT = BE
NEG = -1e30
TM = 512


def _mm_init_body(x_ref, wat_ref, w2a_ref, b2_ref, af_ref, a_ref):
    af = jnp.dot(x_ref[...], wat_ref[...], preferred_element_type=jnp.float32)
    af_ref[...] = af
    a_ref[...] = jnp.dot(af, w2a_ref[...], preferred_element_type=jnp.float32) + b2_ref[...]


def _mm_step_body(af_ref, nei_ref, w1a_ref, w1b_ref, b1_ref, w2a_ref, b2_ref,
                  af_out, a_out):
    x = jnp.dot(af_ref[...], w1a_ref[...], preferred_element_type=jnp.float32)
    x = x + jnp.dot(nei_ref[...], w1b_ref[...], preferred_element_type=jnp.float32)
    x = jnp.maximum(x + b1_ref[...], 0.0)
    af_out[...] = x
    a_out[...] = jnp.dot(x, w2a_ref[...], preferred_element_type=jnp.float32) + b2_ref[...]


def _mm_final_body(af_ref, nei_ref, w1a_ref, w1b_ref, b1_ref, af_out):
    x = jnp.dot(af_ref[...], w1a_ref[...], preferred_element_type=jnp.float32)
    x = x + jnp.dot(nei_ref[...], w1b_ref[...], preferred_element_type=jnp.float32)
    af_out[...] = jnp.maximum(x + b1_ref[...], 0.0)


def _mm_bond_body(bond_ref, w2b_ref, out_ref):
    v = jnp.dot(bond_ref[...], w2b_ref[...], preferred_element_type=jnp.float32)
    row = pl.program_id(0) * TM + lax.broadcasted_iota(jnp.int32, (TM, H), 0)
    out_ref[...] = jnp.where(row < BE, v, NEG)


def _full(shape):
    return pl.BlockSpec(shape, lambda i: (0, 0))


def _rows(w):
    return pl.BlockSpec((TM, w), lambda i: (i, 0))


def _tc_init(xp, wat, w2a, b2):
    return pl.pallas_call(
        _mm_init_body,
        grid=(NP // TM,),
        in_specs=[_rows(128), _full((128, H)), _full((H, H)), _full((1, H))],
        out_specs=[_rows(H), _rows(H)],
        out_shape=[jax.ShapeDtypeStruct((NP, H), jnp.float32),
                   jax.ShapeDtypeStruct((NP, H), jnp.float32)],
    )(xp, wat, w2a, b2)


def _tc_step(af, nei, w1a, w1b, b1, w2a, b2):
    return pl.pallas_call(
        _mm_step_body,
        grid=(NP // TM,),
        in_specs=[_rows(H), _rows(H), _full((H, H)), _full((H, H)), _full((1, H)),
                  _full((H, H)), _full((1, H))],
        out_specs=[_rows(H), _rows(H)],
        out_shape=[jax.ShapeDtypeStruct((NP, H), jnp.float32),
                   jax.ShapeDtypeStruct((NP, H), jnp.float32)],
    )(af, nei, w1a, w1b, b1, w2a, b2)


def _tc_final(af, nei, w1a, w1b, b1):
    return pl.pallas_call(
        _mm_final_body,
        grid=(NP // TM,),
        in_specs=[_rows(H), _rows(H), _full((H, H)), _full((H, H)), _full((1, H))],
        out_specs=_rows(H),
        out_shape=jax.ShapeDtypeStruct((NP, H), jnp.float32),
    )(af, nei, w1a, w1b, b1)


def _tc_bond(bondp, w2b):
    return pl.pallas_call(
        _mm_bond_body,
        grid=(NPE // TM,),
        in_specs=[pl.BlockSpec((TM, 8), lambda i: (i, 0)), _full((8, H))],
        out_specs=_rows(H),
        out_shape=jax.ShapeDtypeStruct((NPE, H), jnp.float32),
    )(bondp, w2b)


def _sc_nei(A, Bv, a_idx, e_idx):
    info = plsc.get_sparse_core_info()
    nc = info.num_cores
    mesh = plsc.VectorSubcoreMesh(core_axis_name="c", subcore_axis_name="s")
    gpw = RPW * MAX_NB
    nch = RPW // CROWS

    @functools.partial(
        pl.kernel,
        mesh=mesh,
        out_type=jax.ShapeDtypeStruct((NP, H), jnp.float32),
        scratch_types=[
            pltpu.VMEM((gpw,), jnp.int32),
            pltpu.VMEM((gpw,), jnp.int32),
            pltpu.VMEM((CG, H), jnp.float32),
            pltpu.VMEM((CG, H), jnp.float32),
            pltpu.VMEM((CG, H), jnp.float32),
            pltpu.VMEM((CG, H), jnp.float32),
            pltpu.VMEM((CROWS, H), jnp.float32),
            pltpu.VMEM((CROWS, H), jnp.float32),
            pltpu.SemaphoreType.DMA,
            pltpu.SemaphoreType.DMA,
        ],
    )
    def k(a_hbm, bv_hbm, ai_hbm, ei_hbm, out_hbm,
          ai_v, ei_v, ga0, gb0, ga1, gb1, oc0, oc1, sg0, sg1):
        wid = lax.axis_index("s") * nc + lax.axis_index("c")
        row_base = wid * RPW
        bufs = ((ga0, gb0, oc0, sg0), (ga1, gb1, oc1, sg1))

        pltpu.sync_copy(ai_hbm.at[pl.ds(row_base * MAX_NB, gpw)], ai_v)
        pltpu.sync_copy(ei_hbm.at[pl.ds(row_base * MAX_NB, gpw)], ei_v)

        def fire(c, b):
            ga, gb, _, sg = bufs[b]
            g0 = c * CG
            for j in range(CG // SUB):
                s = pl.ds(g0 + j * SUB, SUB)
                dsl = pl.ds(j * SUB, SUB)
                pltpu.async_copy(a_hbm.at[ai_v.at[s]], ga.at[dsl], sg)
                pltpu.async_copy(bv_hbm.at[ei_v.at[s]], gb.at[dsl], sg)

        def drain(b):
            ga, gb, _, sg = bufs[b]
            pltpu.make_async_copy(a_hbm.at[ai_v.at[pl.ds(0, CG)]], ga, sg).wait()
            pltpu.make_async_copy(bv_hbm.at[ei_v.at[pl.ds(0, CG)]], gb, sg).wait()

        def compute_store(c, b):
            ga, gb, oc, _ = bufs[b]

            def row(r, c2):
                rk = r * MAX_NB
                for h in range(H // 16):
                    sl = pl.ds(h * 16, 16)
                    acc = jnp.zeros((16,), jnp.float32)
                    for kk in range(MAX_NB):
                        v = ga[rk + kk, sl] + gb[rk + kk, sl]
                        acc = acc + jnp.maximum(v, 0.0)
                    oc[r, sl] = acc
                return c2

            lax.fori_loop(0, CROWS, row, 0)
            pltpu.sync_copy(oc, out_hbm.at[pl.ds(row_base + c * CROWS, CROWS)])

        fire(0, 0)

        def pair(i, carry):
            c0 = 2 * i
            fire(c0 + 1, 1)
            drain(0)
            compute_store(c0, 0)
            fire(c0 + 2, 0)
            drain(1)
            compute_store(c0 + 1, 1)
            return carry

        lax.fori_loop(0, (nch - 1) // 2, pair, 0)
        drain(0)
        compute_store(nch - 1, 0)

    return k(A, Bv, a_idx, e_idx)


def kernel(input_atom, input_bond, atom_graph, bond_graph, num_nbs,
           W_atom, W_U2, b_U2, W_U1, b_U1):
    f32 = jnp.float32
    x = input_atom.reshape(BN, F_ATOM)
    xp = jnp.zeros((NP, 128), f32).at[:BN, :F_ATOM].set(x)
    wat = jnp.zeros((128, H), f32).at[:F_ATOM].set(W_atom)
    bond = input_bond.reshape(BE, F_BOND)
    bondp = jnp.zeros((NPE, 8), f32).at[:BE, :F_BOND].set(bond)
    w2a = W_U2[:H]
    w2b = jnp.zeros((8, H), f32).at[:F_BOND].set(W_U2[H:])
    w1a = W_U1[:H]
    w1b = W_U1[H:]
    b1 = b_U1.reshape(1, H)
    b2 = b_U2.reshape(1, H)

    a_flat = (atom_graph[..., 0] * N + atom_graph[..., 1]).reshape(-1)
    valid = jnp.arange(MAX_NB, dtype=jnp.int32)[None, None, :] < num_nbs[:, :, None]
    e_flat = jnp.where(valid, bond_graph[..., 0] * E + bond_graph[..., 1],
                       SENT).reshape(-1)
    pad_g = NP * MAX_NB - BN * MAX_NB
    a_flat = jnp.concatenate([a_flat, jnp.zeros((pad_g,), jnp.int32)])
    e_flat = jnp.concatenate([e_flat, jnp.full((pad_g,), SENT, jnp.int32)])

    bv = _tc_bond(bondp, w2b)
    af, a_tab = _tc_init(xp, wat, w2a, b2)
    for d in range(DEPTH):
        nei = _sc_nei(a_tab, bv, a_flat, e_flat)
        if d < DEPTH - 1:
            af, a_tab = _tc_step(af, nei, w1a, w1b, b1, w2a, b2)
        else:
            af = _tc_final(af, nei, w1a, w1b, b1)
    return af[:BN].reshape(B, N, H)


# speedup vs baseline: 21.9933x; 21.7158x over previous
import functools

import jax
import jax.numpy as jnp
from jax import lax
from jax.experimental import pallas as pl
from jax.experimental.pallas import tpu as pltpu
from jax.experimental.pallas import tpu_sc as plsc

B, N, E, MAX_NB, H, F_ATOM, F_BOND, DEPTH = 4, 12500, 12500, 10, 128, 89, 5, 3
BN = B * N
BE = B * E
NW = 32
CROWS = 16
SUB = 80
NP = 50688
RPW = NP // NW
CG = CROWS * MAX_NB
NSENT = 2048
NPE = 52224
SENT = BE
NEG = -1e30
TM = 512


def _mm_init_body(x_ref, wat_ref, w2a_ref, b2_ref, af_ref, a_ref):
    af = jnp.dot(x_ref[...], wat_ref[...], preferred_element_type=jnp.float32)
    af_ref[...] = af
    a_ref[...] = jnp.dot(af, w2a_ref[...], preferred_element_type=jnp.float32) + b2_ref[...]


def _mm_step_body(af_ref, nei_ref, w1a_ref, w1b_ref, b1_ref, w2a_ref, b2_ref,
                  af_out, a_out):
    x = jnp.dot(af_ref[...], w1a_ref[...], preferred_element_type=jnp.float32)
    x = x + jnp.dot(nei_ref[...], w1b_ref[...], preferred_element_type=jnp.float32)
    x = jnp.maximum(x + b1_ref[...], 0.0)
    af_out[...] = x
    a_out[...] = jnp.dot(x, w2a_ref[...], preferred_element_type=jnp.float32) + b2_ref[...]


def _mm_final_body(af_ref, nei_ref, w1a_ref, w1b_ref, b1_ref, af_out):
    x = jnp.dot(af_ref[...], w1a_ref[...], preferred_element_type=jnp.float32)
    x = x + jnp.dot(nei_ref[...], w1b_ref[...], preferred_element_type=jnp.float32)
    af_out[...] = jnp.maximum(x + b1_ref[...], 0.0)


def _mm_bond_body(bond_ref, w2b_ref, out_ref):
    v = jnp.dot(bond_ref[...], w2b_ref[...], preferred_element_type=jnp.float32)
    row = pl.program_id(0) * TM + lax.broadcasted_iota(jnp.int32, (TM, H), 0)
    out_ref[...] = jnp.where(row < BE, v, NEG)


def _full(shape):
    return pl.BlockSpec(shape, lambda i: (0, 0))


def _rows(w):
    return pl.BlockSpec((TM, w), lambda i: (i, 0))


def _tc_init(xp, wat, w2a, b2):
    return pl.pallas_call(
        _mm_init_body,
        grid=(NP // TM,),
        in_specs=[_rows(128), _full((128, H)), _full((H, H)), _full((1, H))],
        out_specs=[_rows(H), _rows(H)],
        out_shape=[jax.ShapeDtypeStruct((NP, H), jnp.float32),
                   jax.ShapeDtypeStruct((NP, H), jnp.float32)],
    )(xp, wat, w2a, b2)


def _tc_step(af, nei, w1a, w1b, b1, w2a, b2):
    return pl.pallas_call(
        _mm_step_body,
        grid=(NP // TM,),
        in_specs=[_rows(H), _rows(H), _full((H, H)), _full((H, H)), _full((1, H)),
                  _full((H, H)), _full((1, H))],
        out_specs=[_rows(H), _rows(H)],
        out_shape=[jax.ShapeDtypeStruct((NP, H), jnp.float32),
                   jax.ShapeDtypeStruct((NP, H), jnp.float32)],
    )(af, nei, w1a, w1b, b1, w2a, b2)


def _tc_final(af, nei, w1a, w1b, b1):
    return pl.pallas_call(
        _mm_final_body,
        grid=(NP // TM,),
        in_specs=[_rows(H), _rows(H), _full((H, H)), _full((H, H)), _full((1, H))],
        out_specs=_rows(H),
        out_shape=jax.ShapeDtypeStruct((NP, H), jnp.float32),
    )(af, nei, w1a, w1b, b1)


def _tc_bond(bondp, w2b):
    return pl.pallas_call(
        _mm_bond_body,
        grid=(NPE // TM,),
        in_specs=[pl.BlockSpec((TM, 8), lambda i: (i, 0)), _full((8, H))],
        out_specs=_rows(H),
        out_shape=jax.ShapeDtypeStruct((NPE, H), jnp.float32),
    )(bondp, w2b)


def _sc_nei(A, Bv, a_idx, e_idx):
    info = plsc.get_sparse_core_info()
    nc = info.num_cores
    mesh = plsc.VectorSubcoreMesh(core_axis_name="c", subcore_axis_name="s")
    gpw = RPW * MAX_NB
    nch = RPW // CROWS

    @functools.partial(
        pl.kernel,
        mesh=mesh,
        out_type=jax.ShapeDtypeStruct((NP, H), jnp.float32),
        scratch_types=[
            pltpu.VMEM((gpw,), jnp.int32),
            pltpu.VMEM((gpw,), jnp.int32),
            pltpu.VMEM((CG, H), jnp.float32),
            pltpu.VMEM((CG, H), jnp.float32),
            pltpu.VMEM((CG, H), jnp.float32),
            pltpu.VMEM((CG, H), jnp.float32),
            pltpu.VMEM((CROWS, H), jnp.float32),
            pltpu.VMEM((CROWS, H), jnp.float32),
            pltpu.SemaphoreType.DMA,
            pltpu.SemaphoreType.DMA,
        ],
    )
    def k(a_hbm, bv_hbm, ai_hbm, ei_hbm, out_hbm,
          ai_v, ei_v, ga0, gb0, ga1, gb1, oc0, oc1, sg0, sg1):
        wid = lax.axis_index("s") * nc + lax.axis_index("c")
        row_base = wid * RPW
        bufs = ((ga0, gb0, oc0, sg0), (ga1, gb1, oc1, sg1))

        pltpu.sync_copy(ai_hbm.at[pl.ds(row_base * MAX_NB, gpw)], ai_v)
        pltpu.sync_copy(ei_hbm.at[pl.ds(row_base * MAX_NB, gpw)], ei_v)

        def fire(c, b):
            ga, gb, _, sg = bufs[b]
            g0 = c * CG
            for j in range(CG // SUB):
                s = pl.ds(g0 + j * SUB, SUB)
                dsl = pl.ds(j * SUB, SUB)
                pltpu.async_copy(a_hbm.at[ai_v.at[s]], ga.at[dsl], sg)
                pltpu.async_copy(bv_hbm.at[ei_v.at[s]], gb.at[dsl], sg)

        def drain(b):
            ga, gb, _, sg = bufs[b]
            pltpu.make_async_copy(a_hbm.at[ai_v.at[pl.ds(0, CG)]], ga, sg).wait()
            pltpu.make_async_copy(bv_hbm.at[ei_v.at[pl.ds(0, CG)]], gb, sg).wait()

        def compute_store(c, b):
            ga, gb, oc, _ = bufs[b]

            def row(r, c2):
                rk = r * MAX_NB
                for h in range(H // 16):
                    sl = pl.ds(h * 16, 16)
                    acc = jnp.zeros((16,), jnp.float32)
                    for kk in range(MAX_NB):
                        v = ga[rk + kk, sl] + gb[rk + kk, sl]
                        acc = acc + jnp.maximum(v, 0.0)
                    oc[r, sl] = acc
                return c2

            lax.fori_loop(0, CROWS, row, 0)
            pltpu.sync_copy(oc, out_hbm.at[pl.ds(row_base + c * CROWS, CROWS)])

        fire(0, 0)

        def pair(i, carry):
            c0 = 2 * i
            fire(c0 + 1, 1)
            drain(0)
            compute_store(c0, 0)
            fire(c0 + 2, 0)
            drain(1)
            compute_store(c0 + 1, 1)
            return carry

        lax.fori_loop(0, (nch - 1) // 2, pair, 0)
        drain(0)
        compute_store(nch - 1, 0)

    return k(A, Bv, a_idx, e_idx)


def kernel(input_atom, input_bond, atom_graph, bond_graph, num_nbs,
           W_atom, W_U2, b_U2, W_U1, b_U1):
    f32 = jnp.float32
    x = input_atom.reshape(BN, F_ATOM)
    xp = jnp.zeros((NP, 128), f32).at[:BN, :F_ATOM].set(x)
    wat = jnp.zeros((128, H), f32).at[:F_ATOM].set(W_atom)
    bond = input_bond.reshape(BE, F_BOND)
    bondp = jnp.zeros((NPE, 8), f32).at[:BE, :F_BOND].set(bond)
    w2a = W_U2[:H]
    w2b = jnp.zeros((8, H), f32).at[:F_BOND].set(W_U2[H:])
    w1a = W_U1[:H]
    w1b = W_U1[H:]
    b1 = b_U1.reshape(1, H)
    b2 = b_U2.reshape(1, H)

    a_flat = (atom_graph[..., 0] * N + atom_graph[..., 1]).reshape(-1)
    valid = jnp.arange(MAX_NB, dtype=jnp.int32)[None, None, :] < num_nbs[:, :, None]
    sent = SENT + (jnp.arange(BN * MAX_NB, dtype=jnp.int32) % NSENT)
    e_flat = jnp.where(valid.reshape(-1),
                       (bond_graph[..., 0] * E + bond_graph[..., 1]).reshape(-1),
                       sent)
    pad_g = NP * MAX_NB - BN * MAX_NB
    pad_i = jnp.arange(pad_g, dtype=jnp.int32)
    a_flat = jnp.concatenate([a_flat, pad_i % BN])
    e_flat = jnp.concatenate([e_flat, SENT + pad_i % NSENT])

    bv = _tc_bond(bondp, w2b)
    af, a_tab = _tc_init(xp, wat, w2a, b2)
    for d in range(DEPTH):
        nei = _sc_nei(a_tab, bv, a_flat, e_flat)
        if d < DEPTH - 1:
            af, a_tab = _tc_step(af, nei, w1a, w1b, b1, w2a, b2)
        else:
            af = _tc_final(af, nei, w1a, w1b, b1)
    return af[:BN].reshape(B, N, H)


# speedup vs baseline: 23.4641x; 1.0669x over previous
import functools

import jax
import jax.numpy as jnp
from jax import lax
from jax.experimental import pallas as pl
from jax.experimental.pallas import tpu as pltpu
from jax.experimental.pallas import tpu_sc as plsc

B, N, E, MAX_NB, H, F_ATOM, F_BOND, DEPTH = 4, 12500, 12500, 10, 128, 89, 5, 3
NPAD = 12800
NP = B * NPAD
NEB = 13312
NPE = B * NEB
NSENT = NEB - N
NW = 32
CROWS = 16
SUB = 80
CG = CROWS * MAX_NB
RPW = NP // NW
NCH = RPW // CROWS
NEG = -1e30
TM = 512


def _mm_init_body(x_ref, wat_ref, w2a_ref, b2_ref, af_ref, a_ref):
    af = jnp.dot(x_ref[0], wat_ref[...], preferred_element_type=jnp.float32)
    af_ref[...] = af
    a_ref[...] = jnp.dot(af, w2a_ref[...], preferred_element_type=jnp.float32) + b2_ref[...]


def _mm_step_body(af_ref, nei_ref, w1a_ref, w1b_ref, b1_ref, w2a_ref, b2_ref,
                  af_out, a_out):
    x = jnp.dot(af_ref[...], w1a_ref[...], preferred_element_type=jnp.float32)
    x = x + jnp.dot(nei_ref[...], w1b_ref[...], preferred_element_type=jnp.float32)
    x = jnp.maximum(x + b1_ref[...], 0.0)
    af_out[...] = x
    a_out[...] = jnp.dot(x, w2a_ref[...], preferred_element_type=jnp.float32) + b2_ref[...]


def _mm_final_body(af_ref, nei_ref, w1a_ref, w1b_ref, b1_ref, out_ref):
    x = jnp.dot(af_ref[...], w1a_ref[...], preferred_element_type=jnp.float32)
    x = x + jnp.dot(nei_ref[...], w1b_ref[...], preferred_element_type=jnp.float32)
    out_ref[0] = jnp.maximum(x + b1_ref[...], 0.0)


def _mm_bond_body(bond_ref, w2b_ref, out_ref):
    v = jnp.dot(bond_ref[0], w2b_ref[...], preferred_element_type=jnp.float32)
    row = pl.program_id(1) * TM + lax.broadcasted_iota(jnp.int32, (TM, H), 0)
    out_ref[...] = jnp.where(row < E, v, NEG)


def _full2(shape):
    n = len(shape)

    def im(*_):
        return (0,) * n

    return pl.BlockSpec(shape, im)


def _tc_init(x3, wat, w2a, b2):
    return pl.pallas_call(
        _mm_init_body,
        grid=(B, NPAD // TM),
        in_specs=[pl.BlockSpec((1, TM, F_ATOM), lambda b, j: (b, j, 0)),
                  _full2((F_ATOM, H)), _full2((H, H)), _full2((1, H))],
        out_specs=[pl.BlockSpec((TM, H), lambda b, j: (b * (NPAD // TM) + j, 0)),
                   pl.BlockSpec((TM, H), lambda b, j: (b * (NPAD // TM) + j, 0))],
        out_shape=[jax.ShapeDtypeStruct((NP, H), jnp.float32),
                   jax.ShapeDtypeStruct((NP, H), jnp.float32)],
    )(x3, wat, w2a, b2)


def _tc_bond(bond3, w2b):
    return pl.pallas_call(
        _mm_bond_body,
        grid=(B, NEB // TM),
        in_specs=[pl.BlockSpec((1, TM, F_BOND),
                               lambda b, j: (b, jnp.minimum(j, NPAD // TM - 1), 0)),
                  _full2((F_BOND, H))],
        out_specs=pl.BlockSpec((TM, H), lambda b, j: (b * (NEB // TM) + j, 0)),
        out_shape=jax.ShapeDtypeStruct((NPE, H), jnp.float32),
    )(bond3, w2b)


def _tc_step(af, nei, w1a, w1b, b1, w2a, b2):
    rows = pl.BlockSpec((TM, H), lambda i: (i, 0))
    return pl.pallas_call(
        _mm_step_body,
        grid=(NP // TM,),
        in_specs=[rows, rows, _full2((H, H)), _full2((H, H)), _full2((1, H)),
                  _full2((H, H)), _full2((1, H))],
        out_specs=[rows, rows],
        out_shape=[jax.ShapeDtypeStruct((NP, H), jnp.float32),
                   jax.ShapeDtypeStruct((NP, H), jnp.float32)],
    )(af, nei, w1a, w1b, b1, w2a, b2)


def _tc_final(af, nei, w1a, w1b, b1):
    rows = pl.BlockSpec((TM, H), lambda b, j: (b * (NPAD // TM) + j, 0))
    return pl.pallas_call(
        _mm_final_body,
        grid=(B, NPAD // TM),
        in_specs=[rows, rows, _full2((H, H)), _full2((H, H)), _full2((1, H))],
        out_specs=pl.BlockSpec((1, TM, H), lambda b, j: (b, j, 0)),
        out_shape=jax.ShapeDtypeStruct((B, N, H), jnp.float32),
    )(af, nei, w1a, w1b, b1)


def _sc_nei(A, Bv, a_idx, e_idx):
    info = plsc.get_sparse_core_info()
    nc = info.num_cores
    mesh = plsc.VectorSubcoreMesh(core_axis_name="c", subcore_axis_name="s")
    gpw = RPW * MAX_NB

    @functools.partial(
        pl.kernel,
        mesh=mesh,
        out_type=jax.ShapeDtypeStruct((NP, H), jnp.float32),
        scratch_types=[
            pltpu.VMEM((gpw,), jnp.int32),
            pltpu.VMEM((gpw,), jnp.int32),
            pltpu.VMEM((CG, H), jnp.float32),
            pltpu.VMEM((CG, H), jnp.float32),
            pltpu.VMEM((CG, H), jnp.float32),
            pltpu.VMEM((CG, H), jnp.float32),
            pltpu.VMEM((CROWS, H), jnp.float32),
            pltpu.VMEM((CROWS, H), jnp.float32),
            pltpu.SemaphoreType.DMA,
            pltpu.SemaphoreType.DMA,
        ],
    )
    def k(a_hbm, bv_hbm, ai_hbm, ei_hbm, out_hbm,
          ai_v, ei_v, ga0, gb0, ga1, gb1, oc0, oc1, sg0, sg1):
        wid = lax.axis_index("s") * nc + lax.axis_index("c")
        row_base = wid * RPW
        bufs = ((ga0, gb0, oc0, sg0), (ga1, gb1, oc1, sg1))

        pltpu.sync_copy(ai_hbm.at[pl.ds(row_base * MAX_NB, gpw)], ai_v)
        pltpu.sync_copy(ei_hbm.at[pl.ds(row_base * MAX_NB, gpw)], ei_v)

        def fire(c, b):
            ga, gb, _, sg = bufs[b]
            g0 = c * CG
            for j in range(CG // SUB):
                s = pl.ds(g0 + j * SUB, SUB)
                dsl = pl.ds(j * SUB, SUB)
                pltpu.async_copy(a_hbm.at[ai_v.at[s]], ga.at[dsl], sg)
                pltpu.async_copy(bv_hbm.at[ei_v.at[s]], gb.at[dsl], sg)

        def drain(b):
            ga, gb, _, sg = bufs[b]
            pltpu.make_async_copy(a_hbm.at[ai_v.at[pl.ds(0, CG)]], ga, sg).wait()
            pltpu.make_async_copy(bv_hbm.at[ei_v.at[pl.ds(0, CG)]], gb, sg).wait()

        def compute_store(c, b):
            ga, gb, oc, _ = bufs[b]

            def row(r, c2):
                rk = r * MAX_NB
                for h in range(H // 16):
                    sl = pl.ds(h * 16, 16)
                    acc = jnp.zeros((16,), jnp.float32)
                    for kk in range(MAX_NB):
                        v = ga[rk + kk, sl] + gb[rk + kk, sl]
                        acc = acc + jnp.maximum(v, 0.0)
                    oc[r, sl] = acc
                return c2

            lax.fori_loop(0, CROWS, row, 0)
            pltpu.sync_copy(oc, out_hbm.at[pl.ds(row_base + c * CROWS, CROWS)])

        fire(0, 0)

        def pair(i, carry):
            c0 = 2 * i
            fire(c0 + 1, 1)
            drain(0)
            compute_store(c0, 0)
            fire(c0 + 2, 0)
            drain(1)
            compute_store(c0 + 1, 1)
            return carry

        lax.fori_loop(0, NCH // 2 - 1, pair, 0)
        fire(NCH - 1, 1)
        drain(0)
        compute_store(NCH - 2, 0)
        drain(1)
        compute_store(NCH - 1, 1)

    return k(A, Bv, a_idx, e_idx)


def kernel(input_atom, input_bond, atom_graph, bond_graph, num_nbs,
           W_atom, W_U2, b_U2, W_U1, b_U1):
    w2a = W_U2[:H]
    w2b = W_U2[H:]
    w1a = W_U1[:H]
    w1b = W_U1[H:]
    b1 = b_U1.reshape(1, H)
    b2 = b_U2.reshape(1, H)

    a_core = atom_graph[..., 0] * NPAD + atom_graph[..., 1]
    valid = jnp.arange(MAX_NB, dtype=jnp.int32)[None, None, :] < num_nbs[:, :, None]
    boff_e = jnp.arange(B, dtype=jnp.int32)[:, None, None] * NEB
    spread = (jnp.arange(N * MAX_NB, dtype=jnp.int32) % NSENT).reshape(1, N, MAX_NB)
    e_core = jnp.where(valid, bond_graph[..., 0] * NEB + bond_graph[..., 1],
                       boff_e + N + spread)
    npr = NPAD - N
    boff_a = jnp.arange(B, dtype=jnp.int32)[:, None, None] * NPAD
    pidx = jnp.arange(npr * MAX_NB, dtype=jnp.int32).reshape(1, npr, MAX_NB)
    a_pad = jnp.broadcast_to(boff_a + pidx % N, (B, npr, MAX_NB))
    e_pad = jnp.broadcast_to(boff_e + N + pidx % NSENT, (B, npr, MAX_NB))
    a_flat = jnp.concatenate([a_core, a_pad], axis=1).reshape(-1)
    e_flat = jnp.concatenate([e_core, e_pad], axis=1).reshape(-1)

    bv = _tc_bond(input_bond, w2b)
    af, a_tab = _tc_init(input_atom, W_atom, w2a, b2)
    for d in range(DEPTH):
        nei = _sc_nei(a_tab, bv, a_flat, e_flat)
        if d < DEPTH - 1:
            af, a_tab = _tc_step(af, nei, w1a, w1b, b1, w2a, b2)
        else:
            out = _tc_final(af, nei, w1a, w1b, b1)
    return out
